# baseline (speedup 1.0000x reference)
import functools

import jax
import jax.numpy as jnp
from jax import lax
from jax.experimental import pallas as pl
from jax.experimental.pallas import tpu as pltpu
from jax.experimental.pallas import tpu_sc as plsc

N_NODES = 100000
HIDDEN = 128
NUM_SEGMENTS = 1024

BLOCK_R = 1024
TC_BLOCKS = 42
N_TC = TC_BLOCKS * BLOCK_R
WIN = 128
NWIN = NUM_SEGMENTS // WIN

CHUNK = 128
NUM_WORKERS = 32
N_SC = N_NODES - N_TC
SC_FULL = N_SC // CHUNK
TAIL = N_SC - SC_FULL * CHUNK
K_UNIF = SC_FULL // NUM_WORKERS
NUM_EXTRA = SC_FULL - K_UNIF * NUM_WORKERS
NBUF = 4
ROWS_PER_SUBCORE = NUM_SEGMENTS // 16
LANES = 16


def _sc_pool(x_hbm, ids_hbm, acc_hbm,
             ids_all, extra_ids_v, tail_ids_v, xbuf, shared_acc,
             load_sems, scat_sems, ids_sems):
    cid = lax.axis_index("c")
    sid = lax.axis_index("s")
    wid = sid * 2 + cid
    base_row = N_TC + wid * K_UNIF * CHUNK

    ih = {
        k: pltpu.async_copy(
            ids_hbm.at[pl.ds(base_row + k * CHUNK, CHUNK)],
            ids_all.at[k], ids_sems.at[k])
        for k in range(K_UNIF)
    }

    zbuf = xbuf.at[NBUF]

    @pl.loop(0, ROWS_PER_SUBCORE)
    def _(r):
        for l in range(HIDDEN // LANES):
            zbuf[r, pl.ds(l * LANES, LANES)] = jnp.zeros((LANES,), jnp.float32)

    pltpu.sync_copy(zbuf.at[pl.ds(0, ROWS_PER_SUBCORE)],
                    shared_acc.at[pl.ds(sid * ROWS_PER_SUBCORE,
                                        ROWS_PER_SUBCORE)])
    plsc.subcore_barrier()

    def load(k, b):
        return pltpu.async_copy(
            x_hbm.at[pl.ds(base_row + k * CHUNK, CHUNK)],
            xbuf.at[b], load_sems.at[b])

    lh = {k: load(k, k % NBUF) for k in range(min(NBUF, K_UNIF))}
    sh = {}
    for k in range(K_UNIF):
        b = k % NBUF
        lh[k].wait()
        ih[k].wait()
        sh[k] = pltpu.async_copy(xbuf.at[b], shared_acc.at[ids_all.at[k]],
                                 scat_sems.at[b], add=True)
        if k + NBUF < K_UNIF:
            sh[k].wait()
            lh[k + NBUF] = load(k + NBUF, b)
    for k in range(max(K_UNIF - NBUF, 0), K_UNIF):
        sh[k].wait()

    @pl.when(wid < NUM_EXTRA)
    def _():
        base = N_TC + (K_UNIF * NUM_WORKERS + wid) * CHUNK
        pltpu.sync_copy(ids_hbm.at[pl.ds(base, CHUNK)], extra_ids_v.at[0])
        pltpu.sync_copy(x_hbm.at[pl.ds(base, CHUNK)], xbuf.at[0])
        pltpu.sync_copy(xbuf.at[0], shared_acc.at[extra_ids_v.at[0]], add=True)

    @pl.when(wid == NUM_WORKERS - 1)
    def _():
        base = N_TC + SC_FULL * CHUNK
        pltpu.sync_copy(ids_hbm.at[pl.ds(base, TAIL)], tail_ids_v.at[0])
        pltpu.sync_copy(x_hbm.at[pl.ds(base, TAIL)],
                        xbuf.at[1].at[pl.ds(0, TAIL)])
        pltpu.sync_copy(xbuf.at[1].at[pl.ds(0, TAIL)],
                        shared_acc.at[tail_ids_v.at[0]], add=True)

    plsc.subcore_barrier()

    sl = pl.ds(sid * ROWS_PER_SUBCORE, ROWS_PER_SUBCORE)
    pltpu.sync_copy(shared_acc.at[sl], acc_hbm.at[cid].at[sl])


def _tc_pool(ids_ref, x_ref, out_ref):
    i = pl.program_id(0)

    @pl.when(i == 0)
    def _():
        out_ref[...] = jnp.zeros_like(out_ref)

    ids = ids_ref[:]
    x = x_ref[...].astype(jnp.bfloat16)

    c0 = ids_ref[0] // WIN
    c1 = ids_ref[BLOCK_R - 1] // WIN

    seg = jax.lax.broadcasted_iota(
        jnp.int32, (WIN, BLOCK_R), 0).astype(jnp.bfloat16)

    def body(c, _):
        lids = (ids - c * WIN).astype(jnp.bfloat16)
        one_hot = jnp.where(seg == lids[None, :],
                            jnp.bfloat16(1), jnp.bfloat16(0))
        out_ref[c, :, :] += jnp.dot(
            one_hot, x, preferred_element_type=jnp.float32)
        return 0

    lax.fori_loop(c0, c1 + 1, body, 0)


def _combine(acc_ref, tc_ref, out_ref):
    out_ref[...] = acc_ref[0] + acc_ref[1] + tc_ref[...]


def kernel(node_states, segment_ids):
    ids32 = segment_ids.astype(jnp.int32)

    sc_pool = pl.kernel(
        _sc_pool,
        out_type=jax.ShapeDtypeStruct((2, NUM_SEGMENTS, HIDDEN), jnp.float32),
        mesh=plsc.VectorSubcoreMesh(core_axis_name="c", subcore_axis_name="s"),
        scratch_types=[
            pltpu.VMEM((K_UNIF, CHUNK), jnp.int32),
            pltpu.VMEM((1, CHUNK), jnp.int32),
            pltpu.VMEM((1, TAIL), jnp.int32),
            pltpu.VMEM((NBUF + 1, CHUNK, HIDDEN), jnp.float32),
            pltpu.VMEM_SHARED((NUM_SEGMENTS, HIDDEN), jnp.float32),
            pltpu.SemaphoreType.DMA((NBUF,)),
            pltpu.SemaphoreType.DMA((NBUF,)),
            pltpu.SemaphoreType.DMA((K_UNIF,)),
        ],
    )
    acc = sc_pool(node_states, ids32)

    tc_out = pl.pallas_call(
        _tc_pool,
        grid=(TC_BLOCKS,),
        in_specs=[
            pl.BlockSpec((BLOCK_R,), lambda i: (i,)),
            pl.BlockSpec((BLOCK_R, HIDDEN), lambda i: (i, 0)),
        ],
        out_specs=pl.BlockSpec((NWIN, WIN, HIDDEN), lambda i: (0, 0, 0)),
        out_shape=jax.ShapeDtypeStruct((NWIN, WIN, HIDDEN), jnp.float32),
        compiler_params=pltpu.CompilerParams(
            dimension_semantics=("arbitrary",),
        ),
    )(ids32, node_states)
    tc_out = tc_out.reshape(NUM_SEGMENTS, HIDDEN)

    return pl.pallas_call(
        _combine,
        out_shape=jax.ShapeDtypeStruct((NUM_SEGMENTS, HIDDEN), jnp.float32),
    )(acc, tc_out)


# speedup vs baseline: 1.2377x; 1.2377x over previous
import functools

import jax
import jax.numpy as jnp
from jax import lax
from jax.experimental import pallas as pl
from jax.experimental.pallas import tpu as pltpu
from jax.experimental.pallas import tpu_sc as plsc

N_NODES = 100000
HIDDEN = 128
NUM_SEGMENTS = 1024

BLOCK_R = 2048
TC_BLOCKS = 21
N_TC = TC_BLOCKS * BLOCK_R
WIN = 128
NWIN = NUM_SEGMENTS // WIN

CHUNK = 128
NUM_WORKERS = 32
N_SC = N_NODES - N_TC
SC_FULL = N_SC // CHUNK
TAIL = N_SC - SC_FULL * CHUNK
K_UNIF = SC_FULL // NUM_WORKERS
NUM_EXTRA = SC_FULL - K_UNIF * NUM_WORKERS
NBUF = 4
ROWS_PER_SUBCORE = NUM_SEGMENTS // 16
LANES = 16


def _sc_pool(x_hbm, ids_hbm, acc_hbm,
             ids_all, extra_ids_v, tail_ids_v, xbuf, shared_acc,
             load_sems, scat_sems, ids_sems):
    cid = lax.axis_index("c")
    sid = lax.axis_index("s")
    wid = sid * 2 + cid
    base_row = N_TC + wid * K_UNIF * CHUNK

    ih = {
        k: pltpu.async_copy(
            ids_hbm.at[pl.ds(base_row + k * CHUNK, CHUNK)],
            ids_all.at[k], ids_sems.at[k])
        for k in range(K_UNIF)
    }

    zbuf = xbuf.at[NBUF]

    @pl.loop(0, ROWS_PER_SUBCORE)
    def _(r):
        for l in range(HIDDEN // LANES):
            zbuf[r, pl.ds(l * LANES, LANES)] = jnp.zeros((LANES,), jnp.float32)

    pltpu.sync_copy(zbuf.at[pl.ds(0, ROWS_PER_SUBCORE)],
                    shared_acc.at[pl.ds(sid * ROWS_PER_SUBCORE,
                                        ROWS_PER_SUBCORE)])
    plsc.subcore_barrier()

    def load(k, b):
        return pltpu.async_copy(
            x_hbm.at[pl.ds(base_row + k * CHUNK, CHUNK)],
            xbuf.at[b], load_sems.at[b])

    lh = {k: load(k, k % NBUF) for k in range(min(NBUF, K_UNIF))}
    sh = {}
    for k in range(K_UNIF):
        b = k % NBUF
        lh[k].wait()
        ih[k].wait()
        sh[k] = pltpu.async_copy(xbuf.at[b], shared_acc.at[ids_all.at[k]],
                                 scat_sems.at[b], add=True)
        if k + NBUF < K_UNIF:
            sh[k].wait()
            lh[k + NBUF] = load(k + NBUF, b)
    for k in range(max(K_UNIF - NBUF, 0), K_UNIF):
        sh[k].wait()

    @pl.when(wid < NUM_EXTRA)
    def _():
        base = N_TC + (K_UNIF * NUM_WORKERS + wid) * CHUNK
        pltpu.sync_copy(ids_hbm.at[pl.ds(base, CHUNK)], extra_ids_v.at[0])
        pltpu.sync_copy(x_hbm.at[pl.ds(base, CHUNK)], xbuf.at[0])
        pltpu.sync_copy(xbuf.at[0], shared_acc.at[extra_ids_v.at[0]], add=True)

    @pl.when(wid == NUM_WORKERS - 1)
    def _():
        base = N_TC + SC_FULL * CHUNK
        pltpu.sync_copy(ids_hbm.at[pl.ds(base, TAIL)], tail_ids_v.at[0])
        pltpu.sync_copy(x_hbm.at[pl.ds(base, TAIL)],
                        xbuf.at[1].at[pl.ds(0, TAIL)])
        pltpu.sync_copy(xbuf.at[1].at[pl.ds(0, TAIL)],
                        shared_acc.at[tail_ids_v.at[0]], add=True)

    plsc.subcore_barrier()

    sl = pl.ds(sid * ROWS_PER_SUBCORE, ROWS_PER_SUBCORE)
    pltpu.sync_copy(shared_acc.at[sl], acc_hbm.at[cid].at[sl])


def _tc_pool(ids_ref, x_ref, out_ref):
    i = pl.program_id(0)

    @pl.when(i == 0)
    def _():
        out_ref[...] = jnp.zeros_like(out_ref)

    ids = ids_ref[:]
    x = x_ref[...].astype(jnp.bfloat16)

    c0 = ids_ref[0] // WIN
    c1 = ids_ref[BLOCK_R - 1] // WIN

    seg = jax.lax.broadcasted_iota(
        jnp.int32, (WIN, BLOCK_R), 0).astype(jnp.bfloat16)

    def body(c, _):
        lids = (ids - c * WIN).astype(jnp.bfloat16)
        one_hot = jnp.where(seg == lids[None, :],
                            jnp.bfloat16(1), jnp.bfloat16(0))
        out_ref[c, :, :] += jnp.dot(
            one_hot, x, preferred_element_type=jnp.float32)
        return 0

    lax.fori_loop(c0, c1 + 1, body, 0)


def _combine(acc_ref, tc_ref, out_ref):
    out_ref[...] = acc_ref[0] + acc_ref[1] + tc_ref[...]


def kernel(node_states, segment_ids):
    ids32 = segment_ids.astype(jnp.int32)

    sc_pool = pl.kernel(
        _sc_pool,
        out_type=jax.ShapeDtypeStruct((2, NUM_SEGMENTS, HIDDEN), jnp.float32),
        mesh=plsc.VectorSubcoreMesh(core_axis_name="c", subcore_axis_name="s"),
        scratch_types=[
            pltpu.VMEM((K_UNIF, CHUNK), jnp.int32),
            pltpu.VMEM((1, CHUNK), jnp.int32),
            pltpu.VMEM((1, TAIL), jnp.int32),
            pltpu.VMEM((NBUF + 1, CHUNK, HIDDEN), jnp.float32),
            pltpu.VMEM_SHARED((NUM_SEGMENTS, HIDDEN), jnp.float32),
            pltpu.SemaphoreType.DMA((NBUF,)),
            pltpu.SemaphoreType.DMA((NBUF,)),
            pltpu.SemaphoreType.DMA((K_UNIF,)),
        ],
    )
    acc = sc_pool(node_states, ids32)

    tc_out = pl.pallas_call(
        _tc_pool,
        grid=(TC_BLOCKS,),
        in_specs=[
            pl.BlockSpec((BLOCK_R,), lambda i: (i,)),
            pl.BlockSpec((BLOCK_R, HIDDEN), lambda i: (i, 0)),
        ],
        out_specs=pl.BlockSpec((NWIN, WIN, HIDDEN), lambda i: (0, 0, 0)),
        out_shape=jax.ShapeDtypeStruct((NWIN, WIN, HIDDEN), jnp.float32),
        compiler_params=pltpu.CompilerParams(
            dimension_semantics=("arbitrary",),
        ),
    )(ids32, node_states)
    tc_out = tc_out.reshape(NUM_SEGMENTS, HIDDEN)

    return pl.pallas_call(
        _combine,
        out_shape=jax.ShapeDtypeStruct((NUM_SEGMENTS, HIDDEN), jnp.float32),
    )(acc, tc_out)
